# Initial kernel scaffold; baseline (speedup 1.0000x reference)
import jax
import jax.numpy as jnp
from jax.experimental import pallas as pl


def kernel(keys, queue, ptr):
    raise NotImplementedError("write your pallas kernel here")


# speedup vs baseline: 11.0475x; 11.0475x over previous
import functools

import jax
import jax.numpy as jnp
from jax.experimental import pallas as pl
from jax.experimental.pallas import tpu as pltpu


def _body(ptr_ref, keys_ref, queue_ref, out_ref, *, blk, n_rows, k_rows):
    i = pl.program_id(0)
    s = (i * blk - ptr_ref[0]) % k_rows
    take_keys = s < n_rows

    @pl.when(take_keys)
    def _():
        out_ref[...] = keys_ref[...]

    @pl.when(jnp.logical_not(take_keys))
    def _():
        out_ref[...] = queue_ref[...]


def kernel(keys, queue, ptr):
    n, d = keys.shape
    k = queue.shape[0]
    blk = 2048
    grid = k // blk
    ptr_arr = jnp.asarray(ptr, jnp.int32).reshape((1,))

    def keys_map(i, ptr_ref):
        s = (i * blk - ptr_ref[0]) % k
        return (jnp.where(s < n, s // blk, 0), 0)

    def queue_map(i, ptr_ref):
        return (i, 0)

    grid_spec = pltpu.PrefetchScalarGridSpec(
        num_scalar_prefetch=1,
        grid=(grid,),
        in_specs=[
            pl.BlockSpec((blk, d), keys_map),
            pl.BlockSpec((blk, d), queue_map),
        ],
        out_specs=pl.BlockSpec((blk, d), queue_map),
    )
    return pl.pallas_call(
        functools.partial(_body, blk=blk, n_rows=n, k_rows=k),
        grid_spec=grid_spec,
        out_shape=jax.ShapeDtypeStruct((k, d), queue.dtype),
    )(ptr_arr, keys, queue)


# speedup vs baseline: 12.1340x; 1.0984x over previous
import functools

import jax
import jax.numpy as jnp
from jax.experimental import pallas as pl
from jax.experimental.pallas import tpu as pltpu


def _body(ptr_ref, keys_ref, queue_ref, out_ref, *, blk, n_rows, k_rows):
    i = pl.program_id(0)
    s = (i * blk - ptr_ref[0]) % k_rows
    take_keys = s < n_rows

    @pl.when(take_keys)
    def _():
        out_ref[...] = keys_ref[...]

    @pl.when(jnp.logical_not(take_keys))
    def _():
        out_ref[...] = queue_ref[...]


def kernel(keys, queue, ptr):
    n, d = keys.shape
    k = queue.shape[0]
    blk = 4096
    grid = k // blk
    ptr_arr = jnp.asarray(ptr, jnp.int32).reshape((1,))

    def keys_map(i, ptr_ref):
        s = (i * blk - ptr_ref[0]) % k
        return (jnp.where(s < n, s // blk, 0), 0)

    def queue_map(i, ptr_ref):
        return (i, 0)

    grid_spec = pltpu.PrefetchScalarGridSpec(
        num_scalar_prefetch=1,
        grid=(grid,),
        in_specs=[
            pl.BlockSpec((blk, d), keys_map),
            pl.BlockSpec((blk, d), queue_map),
        ],
        out_specs=pl.BlockSpec((blk, d), queue_map),
    )
    return pl.pallas_call(
        functools.partial(_body, blk=blk, n_rows=n, k_rows=k),
        grid_spec=grid_spec,
        out_shape=jax.ShapeDtypeStruct((k, d), queue.dtype),
    )(ptr_arr, keys, queue)


# speedup vs baseline: 12.1819x; 1.0039x over previous
import functools

import jax
import jax.numpy as jnp
from jax.experimental import pallas as pl
from jax.experimental.pallas import tpu as pltpu


def _body(ptr_ref, keys_ref, queue_ref, out_ref, *, blk, n_rows, k_rows):
    i = pl.program_id(0)
    s = (i * blk - ptr_ref[0]) % k_rows
    take_keys = s < n_rows

    @pl.when(take_keys)
    def _():
        out_ref[...] = keys_ref[...]

    @pl.when(jnp.logical_not(take_keys))
    def _():
        out_ref[...] = queue_ref[...]


def kernel(keys, queue, ptr):
    n, d = keys.shape
    k = queue.shape[0]
    blk = 4096
    grid = k // blk
    ptr_arr = jnp.asarray(ptr, jnp.int32).reshape((1,))

    def keys_map(i, ptr_ref):
        s = (i * blk - ptr_ref[0]) % k
        return (jnp.where(s < n, s // blk, n // blk - 1), 0)

    def queue_map(i, ptr_ref):
        s = (i * blk - ptr_ref[0]) % k
        nxt = jnp.minimum(i + (n - s) // blk, grid - 1)
        return (jnp.where(s < n, nxt, i), 0)

    def out_map(i, ptr_ref):
        return (i, 0)

    grid_spec = pltpu.PrefetchScalarGridSpec(
        num_scalar_prefetch=1,
        grid=(grid,),
        in_specs=[
            pl.BlockSpec((blk, d), keys_map),
            pl.BlockSpec((blk, d), queue_map),
        ],
        out_specs=pl.BlockSpec((blk, d), out_map),
    )
    return pl.pallas_call(
        functools.partial(_body, blk=blk, n_rows=n, k_rows=k),
        grid_spec=grid_spec,
        out_shape=jax.ShapeDtypeStruct((k, d), queue.dtype),
    )(ptr_arr, keys, queue)
